# baseline (speedup 1.0000x reference)
import functools

import numpy as np
import jax
import jax.numpy as jnp
from jax import lax
from jax.experimental import pallas as pl
from jax.experimental.pallas import tpu as pltpu
from jax.experimental.pallas import tpu_sc as plsc

MUL = 16
DIMS = (1, 3, 5)
NPATH = len(DIMS)
OUT_DIM = MUL * sum(DIMS)
SH_DIM = sum(DIMS)
WDIM = NPATH * MUL * MUL
TMPDIM = NPATH * MUL
SCALE = 1.0 / np.sqrt(MUL)

NC = 2
NS = 16
NW = NC * NS
CH = 128

def _build_consts():
    G = np.zeros((MUL, MUL * MUL), np.float32)
    SA = np.zeros((WDIM, OUT_DIM), np.float32)
    Bm = np.zeros((SH_DIM, OUT_DIM), np.float32)
    moff = 0
    shoff = 0
    for p, d in enumerate(DIMS):
        for u in range(MUL):
            for w in range(MUL):
                G[u, u * 16 + w] = 1.0
                for j in range(d):
                    SA[p * 256 + u * 16 + w, moff + w * d + j] = SCALE
        for w in range(MUL):
            for j in range(d):
                Bm[shoff + j, moff + w * d + j] = 1.0
        moff += MUL * d
        shoff += d
    return G, SA, Bm


_G, _SA, _B = _build_consts()


def _worker_range(q, r):
    c = lax.axis_index("c")
    s = lax.axis_index("s")
    w = s * NC + c
    start = w * q + jnp.minimum(w, r)
    cnt = q + jnp.where(w < r, 1, 0)
    return start, cnt


def _gather_body(q, r, node_hbm, col_hbm, x_hbm, idx_v, gbuf):
    start, cnt = _worker_range(q, r)
    pltpu.sync_copy(col_hbm.at[pl.ds(start, q + 1)], idx_v)

    def body(j, carry):
        @pl.when(j < cnt)
        def _():
            pltpu.sync_copy(node_hbm.at[idx_v.at[j]], gbuf)
            pltpu.sync_copy(gbuf, x_hbm.at[pl.ds((start + j) * CH, CH)])
        return carry

    lax.fori_loop(0, q + 1, body, 0)


def _sc_gather(nf128, colp, E, q, r):
    mesh = plsc.VectorSubcoreMesh(core_axis_name="c", subcore_axis_name="s")
    return pl.kernel(
        functools.partial(_gather_body, q, r),
        out_type=jax.ShapeDtypeStruct((E, 128), jnp.bfloat16),
        mesh=mesh,
        scratch_types=[
            pltpu.VMEM((q + 1, CH), jnp.int32),
            pltpu.VMEM((CH, 128), jnp.bfloat16),
        ],
        compiler_params=pltpu.CompilerParams(use_tc_tiling_on_sc=False),
    )(nf128, colp)


def _msg_body(ea_ref, x_ref, sh_ref, w1t_ref, b1_ref, w2t_ref, b2_ref,
              g_ref, sa_ref, bm_ref, out0_ref, out1_ref):
    f32 = jnp.float32
    bf16 = jnp.bfloat16
    h0 = jnp.dot(ea_ref[...], w1t_ref[...], preferred_element_type=f32)
    h0 = h0 + b1_ref[...]
    h = h0 * (1.0 / (1.0 + jnp.exp(-h0)))
    w = jnp.dot(h.astype(bf16), w2t_ref[...], preferred_element_type=f32)
    w = (w + b2_ref[...]).astype(bf16)
    xrep = jnp.dot(x_ref[...], g_ref[...],
                   preferred_element_type=f32).astype(bf16)
    xe = jnp.concatenate([xrep, xrep, xrep], axis=1)
    texp = jnp.dot(xe * w, sa_ref[...], preferred_element_type=f32)
    shexp = jnp.dot(sh_ref[...], bm_ref[...], preferred_element_type=f32)
    msgs = texp * shexp
    out0_ref[...] = msgs[:, :128]
    out1_ref[...] = msgs[:, 128:]


def _tc_messages(edge_attr, x128, sh, W1, b1, W2, b2, EB):
    E = edge_attr.shape[0]
    grid = E // EB
    g128 = np.zeros((128, MUL * MUL), np.float32)
    g128[:MUL] = _G
    consts = (W1.T, b1[None, :], W2.T.astype(jnp.bfloat16), b2[None, :],
              jnp.asarray(g128).astype(jnp.bfloat16),
              jnp.asarray(_SA).astype(jnp.bfloat16), jnp.asarray(_B))

    def eb_spec(d):
        return pl.BlockSpec((EB, d), lambda i: (i, 0))

    def full_spec(a):
        return pl.BlockSpec(a.shape, lambda i: (0,) * a.ndim)

    return pl.pallas_call(
        _msg_body,
        grid=(grid,),
        in_specs=[eb_spec(edge_attr.shape[1]), eb_spec(128), eb_spec(SH_DIM)]
        + [full_spec(cst) for cst in consts],
        out_specs=(eb_spec(128), eb_spec(MUL)),
        out_shape=(jax.ShapeDtypeStruct((E, 128), jnp.float32),
                   jax.ShapeDtypeStruct((E, MUL), jnp.float32)),
        compiler_params=pltpu.CompilerParams(
            vmem_limit_bytes=100 * 1024 * 1024),
    )(edge_attr, x128, sh, *consts)


def _scatter_body(q, r, npt, m0_hbm, m1_hbm, row_hbm, z0_hbm, z1_hbm,
                  p0_hbm, p1_hbm, idx_v, m0buf, m1buf, sh0, sh1):
    c = lax.axis_index("c")
    s = lax.axis_index("s")
    start, cnt = _worker_range(q, r)
    pltpu.sync_copy(z0_hbm, sh0.at[pl.ds(s * npt, npt)])
    pltpu.sync_copy(z1_hbm, sh1.at[pl.ds(s * npt, npt)])
    plsc.subcore_barrier()
    pltpu.sync_copy(row_hbm.at[pl.ds(start, q + 1)], idx_v)

    def body(j, carry):
        @pl.when(j < cnt)
        def _():
            g = (start + j) * CH
            pltpu.sync_copy(m0_hbm.at[pl.ds(g, CH)], m0buf)
            pltpu.sync_copy(m1_hbm.at[pl.ds(g, CH)], m1buf)
            pltpu.sync_copy(m0buf, sh0.at[idx_v.at[j]], add=True)
            pltpu.sync_copy(m1buf, sh1.at[idx_v.at[j]], add=True)
        return carry

    lax.fori_loop(0, q + 1, body, 0)
    plsc.subcore_barrier()
    pltpu.sync_copy(sh0.at[pl.ds(s * npt, npt)],
                    p0_hbm.at[c, pl.ds(s * npt, npt)])
    pltpu.sync_copy(sh1.at[pl.ds(s * npt, npt)],
                    p1_hbm.at[c, pl.ds(s * npt, npt)])


def _sc_scatter(m0, m1, rowp, npad, E, q, r):
    npt = npad // NS
    z0 = jnp.zeros((npt, 128), jnp.float32)
    z1 = jnp.zeros((npt, MUL), jnp.float32)
    mesh = plsc.VectorSubcoreMesh(core_axis_name="c", subcore_axis_name="s")
    return pl.kernel(
        functools.partial(_scatter_body, q, r, npt),
        out_type=(jax.ShapeDtypeStruct((NC, npad, 128), jnp.float32),
                  jax.ShapeDtypeStruct((NC, npad, MUL), jnp.float32)),
        mesh=mesh,
        scratch_types=[
            pltpu.VMEM((q + 1, CH), jnp.int32),
            pltpu.VMEM((CH, 128), jnp.float32),
            pltpu.VMEM((CH, MUL), jnp.float32),
            pltpu.VMEM_SHARED((npad, 128), jnp.float32),
            pltpu.VMEM_SHARED((npad, MUL), jnp.float32),
        ],
        compiler_params=pltpu.CompilerParams(use_tc_tiling_on_sc=False),
    )(m0, m1, rowp, z0, z1)


def _combine_body(p0_ref, p1_ref, nf_ref, wlin_ref, out_ref):
    si = jnp.dot(nf_ref[...], wlin_ref[...],
                 preferred_element_type=jnp.float32) * SCALE
    a0 = p0_ref[0] + p0_ref[1]
    a1 = p1_ref[0] + p1_ref[1]
    pad = jnp.zeros((si.shape[0], 128 - MUL), jnp.float32)
    sip = jnp.concatenate([si, pad], axis=1)
    out_ref[...] = jnp.concatenate([a0 + sip, a1], axis=1)


def _tc_combine(p0, p1, node_feat, Wlin, NB):
    N = node_feat.shape[0]
    grid = N // NB
    return pl.pallas_call(
        _combine_body,
        grid=(grid,),
        in_specs=[
            pl.BlockSpec((NC, NB, 128), lambda i: (0, i, 0)),
            pl.BlockSpec((NC, NB, MUL), lambda i: (0, i, 0)),
            pl.BlockSpec((NB, MUL), lambda i: (i, 0)),
            pl.BlockSpec((MUL, MUL), lambda i: (0, 0)),
        ],
        out_specs=pl.BlockSpec((NB, OUT_DIM), lambda i: (i, 0)),
        out_shape=jax.ShapeDtypeStruct((N, OUT_DIM), jnp.float32),
    )(p0, p1, node_feat, Wlin)


def kernel(node_feat, edge_index, edge_attr, sh, W1, b1, W2, b2, Wlin):
    N = node_feat.shape[0]
    E = edge_attr.shape[0]
    T = E // CH
    q, r = T // NW, T % NW
    rowp = jnp.pad(edge_index[0].reshape(T, CH), ((0, q + 1 + r), (0, 0)))
    colp = jnp.pad(edge_index[1].reshape(T, CH), ((0, q + 1 + r), (0, 0)))
    nf128 = jnp.pad(node_feat.astype(jnp.bfloat16), ((0, 0), (0, 128 - MUL)))
    x128 = _sc_gather(nf128, colp, E, q, r)
    m0, m1 = _tc_messages(edge_attr, x128, sh, W1, b1, W2, b2, EB=4000)
    npad = ((N + NW * 8 - 1) // (NW * 8)) * NW * 8
    p0, p1 = _sc_scatter(m0, m1, rowp, npad, E, q, r)
    return _tc_combine(p0, p1, node_feat, Wlin, NB=1000)


# speedup vs baseline: 1.3576x; 1.3576x over previous
import functools

import numpy as np
import jax
import jax.numpy as jnp
from jax import lax
from jax.experimental import pallas as pl
from jax.experimental.pallas import tpu as pltpu
from jax.experimental.pallas import tpu_sc as plsc

MUL = 16
DIMS = (1, 3, 5)
NPATH = len(DIMS)
OUT_DIM = MUL * sum(DIMS)
SH_DIM = sum(DIMS)
WDIM = NPATH * MUL * MUL
TMPDIM = NPATH * MUL
SCALE = 1.0 / np.sqrt(MUL)

NC = 2
NS = 16
NW = NC * NS
CH = 128

def _build_consts():
    G = np.zeros((MUL, MUL * MUL), np.float32)
    SA = np.zeros((WDIM, OUT_DIM), np.float32)
    Bm = np.zeros((SH_DIM, OUT_DIM), np.float32)
    moff = 0
    shoff = 0
    for p, d in enumerate(DIMS):
        for u in range(MUL):
            for w in range(MUL):
                G[u, u * 16 + w] = 1.0
                for j in range(d):
                    SA[p * 256 + u * 16 + w, moff + w * d + j] = SCALE
        for w in range(MUL):
            for j in range(d):
                Bm[shoff + j, moff + w * d + j] = 1.0
        moff += MUL * d
        shoff += d
    return G, SA, Bm


_G, _SA, _B = _build_consts()


def _worker_range(S, T):
    c = lax.axis_index("c")
    s = lax.axis_index("s")
    w = s * NC + c
    start = w * S
    cnt = jnp.clip(T - start, 0, S)
    return start, cnt


def _gather_body(S, T, node_hbm, col_hbm, x_hbm, idx_v, gbuf, gsem, wsem):
    start, cnt = _worker_range(S, T)
    gc = cnt // 2
    pltpu.sync_copy(col_hbm.at[pl.ds(start, S)], idx_v)

    def g_desc(g, k, p):
        return pltpu.make_async_copy(node_hbm.at[idx_v.at[2 * g + k]],
                                     gbuf.at[p, k], gsem.at[p])

    def w_desc(g, k, p):
        return pltpu.make_async_copy(
            gbuf.at[p, k], x_hbm.at[pl.ds((start + 2 * g + k) * CH, CH)],
            wsem.at[p])

    @pl.when(gc > 0)
    def _():
        for k in range(2):
            g_desc(0, k, 0).start()

    def body(i, carry):
        for p in range(2):
            g = 2 * i + p

            @pl.when(g < gc)
            def _(g=g, p=p):
                for k in range(2):
                    g_desc(g, k, p).wait()

                @pl.when(g >= 1)
                def _():
                    for k in range(2):
                        w_desc(g - 1, k, 1 - p).wait()

                @pl.when(g + 1 < gc)
                def _():
                    for k in range(2):
                        g_desc(g + 1, k, 1 - p).start()

                for k in range(2):
                    w_desc(g, k, p).start()
        return carry

    lax.fori_loop(0, S // 4, body, 0)

    for p in range(2):
        @pl.when((gc > 0) & (lax.rem(gc - 1, 2) == p))
        def _(p=p):
            for k in range(2):
                w_desc(gc - 1, k, p).wait()


def _sc_gather(nf128, colp, E, S, T):
    mesh = plsc.VectorSubcoreMesh(core_axis_name="c", subcore_axis_name="s")
    return pl.kernel(
        functools.partial(_gather_body, S, T),
        out_type=jax.ShapeDtypeStruct((E, 128), jnp.float32),
        mesh=mesh,
        scratch_types=[
            pltpu.VMEM((S, CH), jnp.int32),
            pltpu.VMEM((2, 2, CH, 128), jnp.float32),
            pltpu.SemaphoreType.DMA((2,)),
            pltpu.SemaphoreType.DMA((2,)),
        ],
        compiler_params=pltpu.CompilerParams(use_tc_tiling_on_sc=False),
    )(nf128, colp)


def _msg_body(ea_ref, x_ref, sh_ref, w1t_ref, b1_ref, w2t_ref, b2_ref,
              g_ref, sa_ref, bm_ref, out0_ref, out1_ref):
    f32 = jnp.float32
    bf16 = jnp.bfloat16
    h0 = jnp.dot(ea_ref[...], w1t_ref[...], preferred_element_type=f32)
    h0 = h0 + b1_ref[...]
    h = h0 * (1.0 / (1.0 + jnp.exp(-h0)))
    w = jnp.dot(h.astype(bf16), w2t_ref[...], preferred_element_type=f32)
    w = (w + b2_ref[...]).astype(bf16)
    xrep = jnp.dot(x_ref[...].astype(bf16), g_ref[...],
                   preferred_element_type=f32).astype(bf16)
    xe = jnp.concatenate([xrep, xrep, xrep], axis=1)
    texp = jnp.dot(xe * w, sa_ref[...], preferred_element_type=f32)
    shexp = jnp.dot(sh_ref[...], bm_ref[...], preferred_element_type=f32)
    msgs = texp * shexp
    out0_ref[...] = msgs[:, :128]
    out1_ref[...] = msgs[:, 128:]


def _tc_messages(edge_attr, x128, sh, W1, b1, W2, b2, EB):
    E = edge_attr.shape[0]
    grid = E // EB
    g128 = np.zeros((128, MUL * MUL), np.float32)
    g128[:MUL] = _G
    consts = (W1.T, b1[None, :], W2.T.astype(jnp.bfloat16), b2[None, :],
              jnp.asarray(g128).astype(jnp.bfloat16),
              jnp.asarray(_SA).astype(jnp.bfloat16), jnp.asarray(_B))

    def eb_spec(d):
        return pl.BlockSpec((EB, d), lambda i: (i, 0))

    def full_spec(a):
        return pl.BlockSpec(a.shape, lambda i: (0,) * a.ndim)

    return pl.pallas_call(
        _msg_body,
        grid=(grid,),
        in_specs=[eb_spec(edge_attr.shape[1]), eb_spec(128), eb_spec(SH_DIM)]
        + [full_spec(cst) for cst in consts],
        out_specs=(eb_spec(128), eb_spec(MUL)),
        out_shape=(jax.ShapeDtypeStruct((E, 128), jnp.float32),
                   jax.ShapeDtypeStruct((E, MUL), jnp.float32)),
        compiler_params=pltpu.CompilerParams(
            vmem_limit_bytes=100 * 1024 * 1024),
    )(edge_attr, x128, sh, *consts)


def _scatter_body(S, T, npt, m0_hbm, m1_hbm, row_hbm, z0_hbm, z1_hbm,
                  p0_hbm, p1_hbm, idx_v, m0buf, m1buf, sh0, sh1, lsem, asem):
    c = lax.axis_index("c")
    s = lax.axis_index("s")
    start, cnt = _worker_range(S, T)
    pltpu.sync_copy(z0_hbm, sh0.at[pl.ds(s * npt, npt)])
    pltpu.sync_copy(z1_hbm, sh1.at[pl.ds(s * npt, npt)])
    plsc.subcore_barrier()

    def l_descs(j, p):
        e = (start + j) * CH
        return (pltpu.make_async_copy(m0_hbm.at[pl.ds(e, CH)],
                                      m0buf.at[p], lsem.at[p]),
                pltpu.make_async_copy(m1_hbm.at[pl.ds(e, CH)],
                                      m1buf.at[p], lsem.at[p]),
                pltpu.make_async_copy(row_hbm.at[pl.ds(e, CH)],
                                      idx_v.at[p], lsem.at[p]))

    def a_start(j, p):
        pltpu.async_copy(m0buf.at[p], sh0.at[idx_v.at[p]], asem.at[p],
                         add=True)
        pltpu.async_copy(m1buf.at[p], sh1.at[idx_v.at[p]], asem.at[p],
                         add=True)

    def a_wait(j, p):
        pltpu.make_async_copy(m0buf.at[p], sh0.at[idx_v.at[p]],
                              asem.at[p]).wait()
        pltpu.make_async_copy(m1buf.at[p], sh1.at[idx_v.at[p]],
                              asem.at[p]).wait()

    @pl.when(cnt > 0)
    def _():
        for d in l_descs(0, 0):
            d.start()

    def body(i, carry):
        for p in range(2):
            j = 2 * i + p

            @pl.when(j < cnt)
            def _(j=j, p=p):
                for d in l_descs(j, p):
                    d.wait()

                @pl.when(j >= 1)
                def _():
                    a_wait(j - 1, 1 - p)

                @pl.when(j + 1 < cnt)
                def _():
                    for d in l_descs(j + 1, 1 - p):
                        d.start()

                a_start(j, p)
        return carry

    lax.fori_loop(0, S // 2, body, 0)

    for p in range(2):
        @pl.when((cnt > 0) & (lax.rem(cnt - 1, 2) == p))
        def _(p=p):
            a_wait(cnt - 1, p)

    plsc.subcore_barrier()
    pltpu.sync_copy(sh0.at[pl.ds(s * npt, npt)],
                    p0_hbm.at[c, pl.ds(s * npt, npt)])
    pltpu.sync_copy(sh1.at[pl.ds(s * npt, npt)],
                    p1_hbm.at[c, pl.ds(s * npt, npt)])


def _sc_scatter(m0, m1, rowflat, npad, E, S, T):
    npt = npad // NS
    z0 = jnp.zeros((npt, 128), jnp.float32)
    z1 = jnp.zeros((npt, MUL), jnp.float32)
    mesh = plsc.VectorSubcoreMesh(core_axis_name="c", subcore_axis_name="s")
    return pl.kernel(
        functools.partial(_scatter_body, S, T, npt),
        out_type=(jax.ShapeDtypeStruct((NC, npad, 128), jnp.float32),
                  jax.ShapeDtypeStruct((NC, npad, MUL), jnp.float32)),
        mesh=mesh,
        scratch_types=[
            pltpu.VMEM((2, CH), jnp.int32),
            pltpu.VMEM((2, CH, 128), jnp.float32),
            pltpu.VMEM((2, CH, MUL), jnp.float32),
            pltpu.VMEM_SHARED((npad, 128), jnp.float32),
            pltpu.VMEM_SHARED((npad, MUL), jnp.float32),
            pltpu.SemaphoreType.DMA((2,)),
            pltpu.SemaphoreType.DMA((2,)),
        ],
        compiler_params=pltpu.CompilerParams(use_tc_tiling_on_sc=False),
    )(m0, m1, rowflat, z0, z1)


def _combine_body(p0_ref, p1_ref, nf_ref, wlin_ref, out_ref):
    si = jnp.dot(nf_ref[...], wlin_ref[...],
                 preferred_element_type=jnp.float32) * SCALE
    a0 = p0_ref[0] + p0_ref[1]
    a1 = p1_ref[0] + p1_ref[1]
    pad = jnp.zeros((si.shape[0], 128 - MUL), jnp.float32)
    sip = jnp.concatenate([si, pad], axis=1)
    out_ref[...] = jnp.concatenate([a0 + sip, a1], axis=1)


def _tc_combine(p0, p1, node_feat, Wlin, NB):
    N = node_feat.shape[0]
    grid = N // NB
    return pl.pallas_call(
        _combine_body,
        grid=(grid,),
        in_specs=[
            pl.BlockSpec((NC, NB, 128), lambda i: (0, i, 0)),
            pl.BlockSpec((NC, NB, MUL), lambda i: (0, i, 0)),
            pl.BlockSpec((NB, MUL), lambda i: (i, 0)),
            pl.BlockSpec((MUL, MUL), lambda i: (0, 0)),
        ],
        out_specs=pl.BlockSpec((NB, OUT_DIM), lambda i: (i, 0)),
        out_shape=jax.ShapeDtypeStruct((N, OUT_DIM), jnp.float32),
    )(p0, p1, node_feat, Wlin)


def kernel(node_feat, edge_index, edge_attr, sh, W1, b1, W2, b2, Wlin):
    N = node_feat.shape[0]
    E = edge_attr.shape[0]
    T = E // CH
    S = -(-T // NW)
    S += S % 2
    rowp = jnp.pad(edge_index[0], ((0, (NW * S - T) * CH),))
    colp = jnp.pad(edge_index[1].reshape(T, CH), ((0, NW * S - T), (0, 0)))
    nf128 = jnp.pad(node_feat, ((0, 0), (0, 128 - MUL)))
    x128 = _sc_gather(nf128, colp, E, S, T)
    m0, m1 = _tc_messages(edge_attr, x128, sh, W1, b1, W2, b2, EB=4000)
    npad = ((N + NW * 8 - 1) // (NW * 8)) * NW * 8
    p0, p1 = _sc_scatter(m0, m1, rowp, npad, E, S, T)
    return _tc_combine(p0, p1, node_feat, Wlin, NB=1000)


# speedup vs baseline: 1.3592x; 1.0012x over previous
import functools

import numpy as np
import jax
import jax.numpy as jnp
from jax import lax
from jax.experimental import pallas as pl
from jax.experimental.pallas import tpu as pltpu
from jax.experimental.pallas import tpu_sc as plsc

MUL = 16
DIMS = (1, 3, 5)
NPATH = len(DIMS)
OUT_DIM = MUL * sum(DIMS)
SH_DIM = sum(DIMS)
WDIM = NPATH * MUL * MUL
TMPDIM = NPATH * MUL
SCALE = 1.0 / np.sqrt(MUL)

NC = 2
NS = 16
NW = NC * NS
CH = 128

def _build_consts():
    G = np.zeros((MUL, MUL * MUL), np.float32)
    SA = np.zeros((WDIM, OUT_DIM), np.float32)
    Bm = np.zeros((SH_DIM, OUT_DIM), np.float32)
    moff = 0
    shoff = 0
    for p, d in enumerate(DIMS):
        for u in range(MUL):
            for w in range(MUL):
                G[u, u * 16 + w] = 1.0
                for j in range(d):
                    SA[p * 256 + u * 16 + w, moff + w * d + j] = SCALE
        for w in range(MUL):
            for j in range(d):
                Bm[shoff + j, moff + w * d + j] = 1.0
        moff += MUL * d
        shoff += d
    return G, SA, Bm


_G, _SA, _B = _build_consts()


def _worker_range(S, T):
    c = lax.axis_index("c")
    s = lax.axis_index("s")
    w = s * NC + c
    start = w * S
    cnt = jnp.clip(T - start, 0, S)
    return start, cnt


def _gather_body(S, T, node_hbm, col_hbm, x_hbm, idx_v, gbuf, gsem, wsem):
    start, cnt = _worker_range(S, T)
    gc = cnt // 2
    pltpu.sync_copy(col_hbm.at[pl.ds(start, S)], idx_v)

    def g_desc(g, k, p):
        return pltpu.make_async_copy(node_hbm.at[idx_v.at[2 * g + k]],
                                     gbuf.at[p, k], gsem.at[p])

    def w_desc(g, k, p):
        return pltpu.make_async_copy(
            gbuf.at[p, k], x_hbm.at[pl.ds((start + 2 * g + k) * CH, CH)],
            wsem.at[p])

    @pl.when(gc > 0)
    def _():
        for k in range(2):
            g_desc(0, k, 0).start()

    def body(i, carry):
        for p in range(2):
            g = 2 * i + p

            @pl.when(g < gc)
            def _(g=g, p=p):
                for k in range(2):
                    g_desc(g, k, p).wait()

                @pl.when(g >= 1)
                def _():
                    for k in range(2):
                        w_desc(g - 1, k, 1 - p).wait()

                @pl.when(g + 1 < gc)
                def _():
                    for k in range(2):
                        g_desc(g + 1, k, 1 - p).start()

                for k in range(2):
                    w_desc(g, k, p).start()
        return carry

    lax.fori_loop(0, S // 4, body, 0)

    for p in range(2):
        @pl.when((gc > 0) & (lax.rem(gc - 1, 2) == p))
        def _(p=p):
            for k in range(2):
                w_desc(gc - 1, k, p).wait()


def _sc_gather(nf128, colp, E, S, T):
    mesh = plsc.VectorSubcoreMesh(core_axis_name="c", subcore_axis_name="s")
    return pl.kernel(
        functools.partial(_gather_body, S, T),
        out_type=jax.ShapeDtypeStruct((E, 128), jnp.float32),
        mesh=mesh,
        scratch_types=[
            pltpu.VMEM((S, CH), jnp.int32),
            pltpu.VMEM((2, 2, CH, 128), jnp.float32),
            pltpu.SemaphoreType.DMA((2,)),
            pltpu.SemaphoreType.DMA((2,)),
        ],
        compiler_params=pltpu.CompilerParams(use_tc_tiling_on_sc=True),
    )(nf128, colp)


def _msg_body(ea_ref, x_ref, sh_ref, w1t_ref, b1_ref, w2t_ref, b2_ref,
              g_ref, sa_ref, bm_ref, out0_ref, out1_ref):
    f32 = jnp.float32
    bf16 = jnp.bfloat16
    h0 = jnp.dot(ea_ref[...], w1t_ref[...], preferred_element_type=f32)
    h0 = h0 + b1_ref[...]
    h = h0 * (1.0 / (1.0 + jnp.exp(-h0)))
    w = jnp.dot(h.astype(bf16), w2t_ref[...], preferred_element_type=f32)
    w = (w + b2_ref[...]).astype(bf16)
    xrep = jnp.dot(x_ref[...].astype(bf16), g_ref[...],
                   preferred_element_type=f32).astype(bf16)
    xe = jnp.concatenate([xrep, xrep, xrep], axis=1)
    texp = jnp.dot(xe * w, sa_ref[...], preferred_element_type=f32)
    shexp = jnp.dot(sh_ref[...], bm_ref[...], preferred_element_type=f32)
    msgs = texp * shexp
    out0_ref[...] = msgs[:, :128]
    out1_ref[...] = msgs[:, 128:]


def _tc_messages(edge_attr, x128, sh, W1, b1, W2, b2, EB):
    E = edge_attr.shape[0]
    grid = E // EB
    g128 = np.zeros((128, MUL * MUL), np.float32)
    g128[:MUL] = _G
    consts = (W1.T, b1[None, :], W2.T.astype(jnp.bfloat16), b2[None, :],
              jnp.asarray(g128).astype(jnp.bfloat16),
              jnp.asarray(_SA).astype(jnp.bfloat16), jnp.asarray(_B))

    def eb_spec(d):
        return pl.BlockSpec((EB, d), lambda i: (i, 0))

    def full_spec(a):
        return pl.BlockSpec(a.shape, lambda i: (0,) * a.ndim)

    return pl.pallas_call(
        _msg_body,
        grid=(grid,),
        in_specs=[eb_spec(edge_attr.shape[1]), eb_spec(128), eb_spec(SH_DIM)]
        + [full_spec(cst) for cst in consts],
        out_specs=(eb_spec(128), eb_spec(MUL)),
        out_shape=(jax.ShapeDtypeStruct((E, 128), jnp.float32),
                   jax.ShapeDtypeStruct((E, MUL), jnp.float32)),
        compiler_params=pltpu.CompilerParams(
            vmem_limit_bytes=100 * 1024 * 1024),
    )(edge_attr, x128, sh, *consts)


def _scatter_body(S, T, npt, m0_hbm, m1_hbm, row_hbm, z0_hbm, z1_hbm,
                  p0_hbm, p1_hbm, idx_v, m0buf, m1buf, sh0, sh1, lsem, asem):
    c = lax.axis_index("c")
    s = lax.axis_index("s")
    start, cnt = _worker_range(S, T)
    pltpu.sync_copy(z0_hbm, sh0.at[pl.ds(s * npt, npt)])
    pltpu.sync_copy(z1_hbm, sh1.at[pl.ds(s * npt, npt)])
    plsc.subcore_barrier()

    def l_descs(j, p):
        e = (start + j) * CH
        return (pltpu.make_async_copy(m0_hbm.at[pl.ds(e, CH)],
                                      m0buf.at[p], lsem.at[p]),
                pltpu.make_async_copy(m1_hbm.at[pl.ds(e, CH)],
                                      m1buf.at[p], lsem.at[p]),
                pltpu.make_async_copy(row_hbm.at[pl.ds(e, CH)],
                                      idx_v.at[p], lsem.at[p]))

    def a_start(j, p):
        pltpu.async_copy(m0buf.at[p], sh0.at[idx_v.at[p]], asem.at[p],
                         add=True)
        pltpu.async_copy(m1buf.at[p], sh1.at[idx_v.at[p]], asem.at[p],
                         add=True)

    def a_wait(j, p):
        pltpu.make_async_copy(m0buf.at[p], sh0.at[idx_v.at[p]],
                              asem.at[p]).wait()
        pltpu.make_async_copy(m1buf.at[p], sh1.at[idx_v.at[p]],
                              asem.at[p]).wait()

    @pl.when(cnt > 0)
    def _():
        for d in l_descs(0, 0):
            d.start()

    def body(i, carry):
        for p in range(2):
            j = 2 * i + p

            @pl.when(j < cnt)
            def _(j=j, p=p):
                for d in l_descs(j, p):
                    d.wait()

                @pl.when(j >= 1)
                def _():
                    a_wait(j - 1, 1 - p)

                @pl.when(j + 1 < cnt)
                def _():
                    for d in l_descs(j + 1, 1 - p):
                        d.start()

                a_start(j, p)
        return carry

    lax.fori_loop(0, S // 2, body, 0)

    for p in range(2):
        @pl.when((cnt > 0) & (lax.rem(cnt - 1, 2) == p))
        def _(p=p):
            a_wait(cnt - 1, p)

    plsc.subcore_barrier()
    pltpu.sync_copy(sh0.at[pl.ds(s * npt, npt)],
                    p0_hbm.at[c, pl.ds(s * npt, npt)])
    pltpu.sync_copy(sh1.at[pl.ds(s * npt, npt)],
                    p1_hbm.at[c, pl.ds(s * npt, npt)])


def _sc_scatter(m0, m1, rowflat, npad, E, S, T):
    npt = npad // NS
    z0 = jnp.zeros((npt, 128), jnp.float32)
    z1 = jnp.zeros((npt, MUL), jnp.float32)
    mesh = plsc.VectorSubcoreMesh(core_axis_name="c", subcore_axis_name="s")
    return pl.kernel(
        functools.partial(_scatter_body, S, T, npt),
        out_type=(jax.ShapeDtypeStruct((NC, npad, 128), jnp.float32),
                  jax.ShapeDtypeStruct((NC, npad, MUL), jnp.float32)),
        mesh=mesh,
        scratch_types=[
            pltpu.VMEM((2, CH), jnp.int32),
            pltpu.VMEM((2, CH, 128), jnp.float32),
            pltpu.VMEM((2, CH, MUL), jnp.float32),
            pltpu.VMEM_SHARED((npad, 128), jnp.float32),
            pltpu.VMEM_SHARED((npad, MUL), jnp.float32),
            pltpu.SemaphoreType.DMA((2,)),
            pltpu.SemaphoreType.DMA((2,)),
        ],
        compiler_params=pltpu.CompilerParams(use_tc_tiling_on_sc=False),
    )(m0, m1, rowflat, z0, z1)


def _combine_body(p0_ref, p1_ref, nf_ref, wlin_ref, out_ref):
    si = jnp.dot(nf_ref[...], wlin_ref[...],
                 preferred_element_type=jnp.float32) * SCALE
    a0 = p0_ref[0] + p0_ref[1]
    a1 = p1_ref[0] + p1_ref[1]
    pad = jnp.zeros((si.shape[0], 128 - MUL), jnp.float32)
    sip = jnp.concatenate([si, pad], axis=1)
    out_ref[...] = jnp.concatenate([a0 + sip, a1], axis=1)


def _tc_combine(p0, p1, node_feat, Wlin, NB):
    N = node_feat.shape[0]
    grid = N // NB
    return pl.pallas_call(
        _combine_body,
        grid=(grid,),
        in_specs=[
            pl.BlockSpec((NC, NB, 128), lambda i: (0, i, 0)),
            pl.BlockSpec((NC, NB, MUL), lambda i: (0, i, 0)),
            pl.BlockSpec((NB, MUL), lambda i: (i, 0)),
            pl.BlockSpec((MUL, MUL), lambda i: (0, 0)),
        ],
        out_specs=pl.BlockSpec((NB, OUT_DIM), lambda i: (i, 0)),
        out_shape=jax.ShapeDtypeStruct((N, OUT_DIM), jnp.float32),
    )(p0, p1, node_feat, Wlin)


def kernel(node_feat, edge_index, edge_attr, sh, W1, b1, W2, b2, Wlin):
    N = node_feat.shape[0]
    E = edge_attr.shape[0]
    T = E // CH
    S = -(-T // NW)
    S += S % 2
    rowp = jnp.pad(edge_index[0], ((0, (NW * S - T) * CH),))
    colp = jnp.pad(edge_index[1].reshape(T, CH), ((0, NW * S - T), (0, 0)))
    nf128 = jnp.pad(node_feat, ((0, 0), (0, 128 - MUL)))
    x128 = _sc_gather(nf128, colp, E, S, T)
    m0, m1 = _tc_messages(edge_attr, x128, sh, W1, b1, W2, b2, EB=4000)
    npad = ((N + NW * 8 - 1) // (NW * 8)) * NW * 8
    p0, p1 = _sc_scatter(m0, m1, rowp, npad, E, S, T)
    return _tc_combine(p0, p1, node_feat, Wlin, NB=1000)


# speedup vs baseline: 1.5560x; 1.1447x over previous
import functools

import numpy as np
import jax
import jax.numpy as jnp
from jax import lax
from jax.experimental import pallas as pl
from jax.experimental.pallas import tpu as pltpu
from jax.experimental.pallas import tpu_sc as plsc

MUL = 16
DIMS = (1, 3, 5)
NPATH = len(DIMS)
OUT_DIM = MUL * sum(DIMS)
SH_DIM = sum(DIMS)
WDIM = NPATH * MUL * MUL
TMPDIM = NPATH * MUL
SCALE = 1.0 / np.sqrt(MUL)

NC = 2
NS = 16
NW = NC * NS
CH = 128

def _build_consts():
    G = np.zeros((MUL, MUL * MUL), np.float32)
    SA = np.zeros((WDIM, OUT_DIM), np.float32)
    Bm = np.zeros((SH_DIM, OUT_DIM), np.float32)
    moff = 0
    shoff = 0
    for p, d in enumerate(DIMS):
        for u in range(MUL):
            for w in range(MUL):
                G[u, u * 16 + w] = 1.0
                for j in range(d):
                    SA[p * 256 + u * 16 + w, moff + w * d + j] = SCALE
        for w in range(MUL):
            for j in range(d):
                Bm[shoff + j, moff + w * d + j] = 1.0
        moff += MUL * d
        shoff += d
    return G, SA, Bm


_G, _SA, _B = _build_consts()


def _worker_range(S, T):
    c = lax.axis_index("c")
    s = lax.axis_index("s")
    w = s * NC + c
    start = w * S
    cnt = jnp.clip(T - start, 0, S)
    return start, cnt


def _gather_body(S, T, node_hbm, col_hbm, x_hbm, idx_v, gbuf, gsem, wsem):
    start, cnt = _worker_range(S, T)
    gc = cnt // 2
    pltpu.sync_copy(col_hbm.at[pl.ds(start, S)], idx_v)

    def g_desc(g, k, p):
        return pltpu.make_async_copy(node_hbm.at[idx_v.at[2 * g + k]],
                                     gbuf.at[p, k], gsem.at[p])

    def w_desc(g, k, p):
        return pltpu.make_async_copy(
            gbuf.at[p, k], x_hbm.at[pl.ds((start + 2 * g + k) * CH, CH)],
            wsem.at[p])

    @pl.when(gc > 0)
    def _():
        for k in range(2):
            g_desc(0, k, 0).start()

    def body(i, carry):
        for p in range(2):
            g = 2 * i + p

            @pl.when(g < gc)
            def _(g=g, p=p):
                for k in range(2):
                    g_desc(g, k, p).wait()

                @pl.when(g >= 1)
                def _():
                    for k in range(2):
                        w_desc(g - 1, k, 1 - p).wait()

                @pl.when(g + 1 < gc)
                def _():
                    for k in range(2):
                        g_desc(g + 1, k, 1 - p).start()

                for k in range(2):
                    w_desc(g, k, p).start()
        return carry

    lax.fori_loop(0, S // 4, body, 0)

    for p in range(2):
        @pl.when((gc > 0) & (lax.rem(gc - 1, 2) == p))
        def _(p=p):
            for k in range(2):
                w_desc(gc - 1, k, p).wait()


def _sc_gather(nf128, colp, E, S, T):
    mesh = plsc.VectorSubcoreMesh(core_axis_name="c", subcore_axis_name="s")
    return pl.kernel(
        functools.partial(_gather_body, S, T),
        out_type=jax.ShapeDtypeStruct((E, 128), jnp.float32),
        mesh=mesh,
        scratch_types=[
            pltpu.VMEM((S, CH), jnp.int32),
            pltpu.VMEM((2, 2, CH, 128), jnp.float32),
            pltpu.SemaphoreType.DMA((2,)),
            pltpu.SemaphoreType.DMA((2,)),
        ],
        compiler_params=pltpu.CompilerParams(use_tc_tiling_on_sc=True),
    )(nf128, colp)


def _msg_body(eat_ref, x_ref, sht_ref, w1t_ref, b1_ref, w2t_ref, b2_ref,
              g_ref, sa_ref, bm_ref, out0_ref, out1_ref):
    f32 = jnp.float32
    bf16 = jnp.bfloat16
    dn0 = (((0,), (0,)), ((), ()))
    h0 = lax.dot_general(eat_ref[...], w1t_ref[...], dn0,
                         preferred_element_type=f32)
    h0 = h0 + b1_ref[...]
    h = h0 * (1.0 / (1.0 + jnp.exp(-h0)))
    w = jnp.dot(h.astype(bf16), w2t_ref[...], preferred_element_type=f32)
    w = (w + b2_ref[...]).astype(bf16)
    xrep = jnp.dot(x_ref[...].astype(bf16), g_ref[...],
                   preferred_element_type=f32).astype(bf16)
    xe = jnp.concatenate([xrep, xrep, xrep], axis=1)
    texp = jnp.dot(xe * w, sa_ref[...], preferred_element_type=f32)
    shexp = lax.dot_general(sht_ref[...], bm_ref[...], dn0,
                            preferred_element_type=f32)
    msgs = texp * shexp
    out0_ref[...] = msgs[:, :128]
    out1_ref[...] = msgs[:, 128:]


def _tc_messages(edge_attr, x128, sh, W1, b1, W2, b2, EB):
    E = edge_attr.shape[0]
    grid = E // EB
    g128 = np.zeros((128, MUL * MUL), np.float32)
    g128[:MUL] = _G
    consts = (W1.T, b1[None, :], W2.T.astype(jnp.bfloat16), b2[None, :],
              jnp.asarray(g128).astype(jnp.bfloat16),
              jnp.asarray(_SA).astype(jnp.bfloat16), jnp.asarray(_B))

    def eb_spec(d):
        return pl.BlockSpec((EB, d), lambda i: (i, 0))

    def ebt_spec(d):
        return pl.BlockSpec((d, EB), lambda i: (0, i))

    def full_spec(a):
        return pl.BlockSpec(a.shape, lambda i: (0,) * a.ndim)

    return pl.pallas_call(
        _msg_body,
        grid=(grid,),
        in_specs=[ebt_spec(edge_attr.shape[1]), eb_spec(128),
                  ebt_spec(SH_DIM)]
        + [full_spec(cst) for cst in consts],
        out_specs=(eb_spec(128), eb_spec(MUL)),
        out_shape=(jax.ShapeDtypeStruct((E, 128), jnp.float32),
                   jax.ShapeDtypeStruct((E, MUL), jnp.float32)),
        compiler_params=pltpu.CompilerParams(
            vmem_limit_bytes=100 * 1024 * 1024),
    )(edge_attr.T, x128, sh.T, *consts)


def _scatter_body(S, T, npt, m0_hbm, m1_hbm, row_hbm, z0_hbm, z1_hbm,
                  p0_hbm, p1_hbm, idx_v, m0buf, m1buf, sh0, sh1, lsem, asem):
    c = lax.axis_index("c")
    s = lax.axis_index("s")
    start, cnt = _worker_range(S, T)
    pltpu.sync_copy(z0_hbm, sh0.at[pl.ds(s * npt, npt)])
    pltpu.sync_copy(z1_hbm, sh1.at[pl.ds(s * npt, npt)])
    plsc.subcore_barrier()

    def l_descs(j, p):
        e = (start + j) * CH
        return (pltpu.make_async_copy(m0_hbm.at[pl.ds(e, CH)],
                                      m0buf.at[p], lsem.at[p]),
                pltpu.make_async_copy(m1_hbm.at[pl.ds(e, CH)],
                                      m1buf.at[p], lsem.at[p]),
                pltpu.make_async_copy(row_hbm.at[pl.ds(e, CH)],
                                      idx_v.at[p], lsem.at[p]))

    def a_start(j, p):
        pltpu.async_copy(m0buf.at[p], sh0.at[idx_v.at[p]], asem.at[p],
                         add=True)
        pltpu.async_copy(m1buf.at[p], sh1.at[idx_v.at[p]], asem.at[p],
                         add=True)

    def a_wait(j, p):
        pltpu.make_async_copy(m0buf.at[p], sh0.at[idx_v.at[p]],
                              asem.at[p]).wait()
        pltpu.make_async_copy(m1buf.at[p], sh1.at[idx_v.at[p]],
                              asem.at[p]).wait()

    @pl.when(cnt > 0)
    def _():
        for d in l_descs(0, 0):
            d.start()

    def body(i, carry):
        for p in range(2):
            j = 2 * i + p

            @pl.when(j < cnt)
            def _(j=j, p=p):
                for d in l_descs(j, p):
                    d.wait()

                @pl.when(j >= 1)
                def _():
                    a_wait(j - 1, 1 - p)

                @pl.when(j + 1 < cnt)
                def _():
                    for d in l_descs(j + 1, 1 - p):
                        d.start()

                a_start(j, p)
        return carry

    lax.fori_loop(0, S // 2, body, 0)

    for p in range(2):
        @pl.when((cnt > 0) & (lax.rem(cnt - 1, 2) == p))
        def _(p=p):
            a_wait(cnt - 1, p)

    plsc.subcore_barrier()
    pltpu.sync_copy(sh0.at[pl.ds(s * npt, npt)],
                    p0_hbm.at[c, pl.ds(s * npt, npt)])
    pltpu.sync_copy(sh1.at[pl.ds(s * npt, npt)],
                    p1_hbm.at[c, pl.ds(s * npt, npt)])


def _sc_scatter(m0, m1, rowflat, npad, E, S, T):
    npt = npad // NS
    z0 = jnp.zeros((npt, 128), jnp.float32)
    z1 = jnp.zeros((npt, MUL), jnp.float32)
    mesh = plsc.VectorSubcoreMesh(core_axis_name="c", subcore_axis_name="s")
    return pl.kernel(
        functools.partial(_scatter_body, S, T, npt),
        out_type=(jax.ShapeDtypeStruct((NC, npad, 128), jnp.float32),
                  jax.ShapeDtypeStruct((NC, npad, MUL), jnp.float32)),
        mesh=mesh,
        scratch_types=[
            pltpu.VMEM((2, CH), jnp.int32),
            pltpu.VMEM((2, CH, 128), jnp.float32),
            pltpu.VMEM((2, CH, MUL), jnp.float32),
            pltpu.VMEM_SHARED((npad, 128), jnp.float32),
            pltpu.VMEM_SHARED((npad, MUL), jnp.float32),
            pltpu.SemaphoreType.DMA((2,)),
            pltpu.SemaphoreType.DMA((2,)),
        ],
        compiler_params=pltpu.CompilerParams(use_tc_tiling_on_sc=False),
    )(m0, m1, rowflat, z0, z1)


def _combine_body(p0_ref, p1_ref, nf_ref, wlin_ref, out_ref):
    si = jnp.dot(nf_ref[...], wlin_ref[...],
                 preferred_element_type=jnp.float32) * SCALE
    a0 = p0_ref[0] + p0_ref[1]
    a1 = p1_ref[0] + p1_ref[1]
    pad = jnp.zeros((si.shape[0], 128 - MUL), jnp.float32)
    sip = jnp.concatenate([si, pad], axis=1)
    out_ref[...] = jnp.concatenate([a0 + sip, a1], axis=1)


def _tc_combine(p0, p1, node_feat, Wlin, NB):
    N = node_feat.shape[0]
    grid = N // NB
    return pl.pallas_call(
        _combine_body,
        grid=(grid,),
        in_specs=[
            pl.BlockSpec((NC, NB, 128), lambda i: (0, i, 0)),
            pl.BlockSpec((NC, NB, MUL), lambda i: (0, i, 0)),
            pl.BlockSpec((NB, MUL), lambda i: (i, 0)),
            pl.BlockSpec((MUL, MUL), lambda i: (0, 0)),
        ],
        out_specs=pl.BlockSpec((NB, OUT_DIM), lambda i: (i, 0)),
        out_shape=jax.ShapeDtypeStruct((N, OUT_DIM), jnp.float32),
    )(p0, p1, node_feat, Wlin)


def kernel(node_feat, edge_index, edge_attr, sh, W1, b1, W2, b2, Wlin):
    N = node_feat.shape[0]
    E = edge_attr.shape[0]
    T = E // CH
    S = -(-T // NW)
    S += S % 2
    rowp = jnp.pad(edge_index[0], ((0, (NW * S - T) * CH),))
    colp = jnp.pad(edge_index[1].reshape(T, CH), ((0, NW * S - T), (0, 0)))
    nf128 = jnp.pad(node_feat, ((0, 0), (0, 128 - MUL)))
    x128 = _sc_gather(nf128, colp, E, S, T)
    m0, m1 = _tc_messages(edge_attr, x128, sh, W1, b1, W2, b2, EB=6400)
    npad = ((N + NW * 8 - 1) // (NW * 8)) * NW * 8
    p0, p1 = _sc_scatter(m0, m1, rowp, npad, E, S, T)
    return _tc_combine(p0, p1, node_feat, Wlin, NB=1000)
